# baseline (speedup 1.0000x reference)
import jax
import jax.numpy as jnp
from jax import lax
from jax.experimental import pallas as pl
from jax.experimental.pallas import tpu as pltpu

B, S, D_MODEL, D_FEAT = 4, 4096, 2048, 256
TARGET_SNR = 0.3
LN_EPS = 1e-5
BAND = 8


def _graft_body(last_ref, x_ref, ff_ref, g_ref, beta_ref, w_ref, bias_ref,
                out_ref):
    b = pl.program_id(0)
    off = lax.rem(last_ref[b], BAND)

    ff = ff_ref[0]
    mean = jnp.mean(ff, axis=-1, keepdims=True)
    cent = ff - mean
    var = jnp.mean(cent * cent, axis=-1, keepdims=True)
    ln = cent * lax.rsqrt(var + LN_EPS) * g_ref[...] + beta_ref[...]

    proj = lax.dot_general(ln, w_ref[...], (((1,), (1,)), ((), ())),
                           preferred_element_type=jnp.float32)
    proj = proj + bias_ref[...]

    nrm = jnp.sqrt(jnp.sum(proj * proj, axis=-1, keepdims=True))
    direction = proj / jnp.maximum(nrm, 1e-12)

    host = x_ref[0, pl.ds(off, 1), :]
    rms = jnp.sqrt(jnp.mean(host * host, axis=-1, keepdims=True))
    upd = direction * (rms * TARGET_SNR)

    out_ref[...] = x_ref[...]
    out_ref[0, pl.ds(off, 1), :] = host + upd


def kernel(x, faculty_features, ln_gamma, ln_beta, W, b, token_ids,
           last_indices):
    del token_ids
    last = last_indices.astype(jnp.int32)

    grid_spec = pltpu.PrefetchScalarGridSpec(
        num_scalar_prefetch=1,
        grid=(B,),
        in_specs=[
            pl.BlockSpec((1, BAND, D_MODEL),
                         lambda i, last_ref: (i, last_ref[i] // BAND, 0)),
            pl.BlockSpec((1, 1, D_FEAT), lambda i, last_ref: (i, 0, 0)),
            pl.BlockSpec((1, D_FEAT), lambda i, last_ref: (0, 0)),
            pl.BlockSpec((1, D_FEAT), lambda i, last_ref: (0, 0)),
            pl.BlockSpec((D_MODEL, D_FEAT), lambda i, last_ref: (0, 0)),
            pl.BlockSpec((1, D_MODEL), lambda i, last_ref: (0, 0)),
        ],
        out_specs=pl.BlockSpec((1, BAND, D_MODEL),
                               lambda i, last_ref: (i, last_ref[i] // BAND, 0)),
    )

    return pl.pallas_call(
        _graft_body,
        grid_spec=grid_spec,
        out_shape=jax.ShapeDtypeStruct((B, S, D_MODEL), jnp.float32),
        input_output_aliases={1: 0},
    )(last, x, faculty_features.reshape(B, 1, D_FEAT),
      ln_gamma.reshape(1, D_FEAT),
      ln_beta.reshape(1, D_FEAT), W, b.reshape(1, D_MODEL))


# speedup vs baseline: 1.0235x; 1.0235x over previous
import jax
import jax.numpy as jnp
from jax import lax
from jax.experimental import pallas as pl
from jax.experimental.pallas import tpu as pltpu

B, S, D_MODEL, D_FEAT = 4, 4096, 2048, 256
TARGET_SNR = 0.3
LN_EPS = 1e-5
BS = 512


def _body(last_ref, x_ref, ff_ref, g_ref, beta_ref, w_ref, bias_ref, out_ref):
    b = pl.program_id(0)
    j = pl.program_id(1)
    last = last_ref[b]
    jb = last // BS
    off = lax.rem(last, BS)

    out_ref[...] = x_ref[...]

    @pl.when(j == jb)
    def _():
        ff = ff_ref[0]
        mean = jnp.mean(ff, axis=-1, keepdims=True)
        cent = ff - mean
        var = jnp.mean(cent * cent, axis=-1, keepdims=True)
        ln = cent * lax.rsqrt(var + LN_EPS) * g_ref[...] + beta_ref[...]
        proj = lax.dot_general(ln, w_ref[...], (((1,), (1,)), ((), ())),
                               preferred_element_type=jnp.float32)
        proj = proj + bias_ref[...]
        nrm = jnp.sqrt(jnp.sum(proj * proj, axis=-1, keepdims=True))
        direction = proj / jnp.maximum(nrm, 1e-12)
        host = x_ref[0, pl.ds(off, 1), :]
        rms = jnp.sqrt(jnp.mean(host * host, axis=-1, keepdims=True))
        out_ref[0, pl.ds(off, 1), :] = host + direction * (rms * TARGET_SNR)


def kernel(x, faculty_features, ln_gamma, ln_beta, W, b, token_ids,
           last_indices):
    del token_ids
    last = last_indices.astype(jnp.int32)

    grid_spec = pltpu.PrefetchScalarGridSpec(
        num_scalar_prefetch=1,
        grid=(B, S // BS),
        in_specs=[
            pl.BlockSpec((1, BS, D_MODEL), lambda i, j, last_ref: (i, j, 0)),
            pl.BlockSpec((1, 1, D_FEAT), lambda i, j, last_ref: (i, 0, 0)),
            pl.BlockSpec((1, D_FEAT), lambda i, j, last_ref: (0, 0)),
            pl.BlockSpec((1, D_FEAT), lambda i, j, last_ref: (0, 0)),
            pl.BlockSpec((D_MODEL, D_FEAT), lambda i, j, last_ref: (0, 0)),
            pl.BlockSpec((1, D_MODEL), lambda i, j, last_ref: (0, 0)),
        ],
        out_specs=pl.BlockSpec((1, BS, D_MODEL),
                               lambda i, j, last_ref: (i, j, 0)),
    )

    return pl.pallas_call(
        _body,
        grid_spec=grid_spec,
        out_shape=jax.ShapeDtypeStruct((B, S, D_MODEL), jnp.float32),
    )(last, x, faculty_features.reshape(B, 1, D_FEAT),
      ln_gamma.reshape(1, D_FEAT), ln_beta.reshape(1, D_FEAT), W,
      b.reshape(1, D_MODEL))
